# baseline (speedup 1.0000x reference)
import functools

import jax
import jax.numpy as jnp
from jax import lax
from jax.experimental import pallas as pl
from jax.experimental.pallas import tpu as pltpu
from jax.experimental.pallas import tpu_sc as plsc

NC = 2
NS = 16
NW = NC * NS
L = 16

NEG_SLOPE = 0.2


def _leaky(z):
    return jnp.where(z >= 0, z, z * NEG_SLOPE)


def _magic_div(divisor, max_d):
    for s in range(18, 31):
        m = -(-(1 << s) // divisor)
        if all((d * m) >> s == d // divisor for d in range(0, max_d, 7)) and \
           all((d * m) >> s == d // divisor
               for d in range(0, max_d, divisor)) and \
           all(((d - 1) * m) >> s == (d - 1) // divisor
               for d in range(divisor, max_d, divisor)):
            if (max_d - 1) * m < 2 ** 31:
                return m, s
    raise ValueError("no magic divider found")


def _tcm0_body(x_ref, w_ref, a8_ref, h_ref, s8_ref, m_ref):
    i = pl.program_id(0)
    h = jnp.dot(x_ref[...], w_ref[...], preferred_element_type=jnp.float32)
    h_ref[...] = h
    s8 = jnp.dot(h, a8_ref[...], preferred_element_type=jnp.float32)
    s8_ref[...] = s8
    bm = jnp.full((1, 8), jnp.max(s8[:, 0]), jnp.float32)

    @pl.when(i == 0)
    def _():
        m_ref[...] = bm

    @pl.when(i > 0)
    def _():
        m_ref[...] = jnp.maximum(m_ref[...], bm)


def _tcmA_body(p_ref, b_ref, w_ref, a8_ref, h_ref, s8_ref, m_ref):
    i = pl.program_id(0)
    x = jax.nn.relu(p_ref[...] + b_ref[...])
    h = jnp.dot(x, w_ref[...], preferred_element_type=jnp.float32)
    h_ref[...] = h
    s8 = jnp.dot(h, a8_ref[...], preferred_element_type=jnp.float32)
    s8_ref[...] = s8
    bm = jnp.full((1, 8), jnp.max(s8[:, 0]), jnp.float32)

    @pl.when(i == 0)
    def _():
        m_ref[...] = bm

    @pl.when(i > 0)
    def _():
        m_ref[...] = jnp.maximum(m_ref[...], bm)


def _mlp_body(p_ref, b_ref, wm1_ref, bm1_ref, wm2_ref, bm2_ref, o_ref):
    x = jax.nn.relu(p_ref[...] + b_ref[...])
    t = jax.nn.relu(jnp.dot(x, wm1_ref[...], preferred_element_type=jnp.float32)
                    + bm1_ref[...])
    o_ref[...] = jnp.dot(t, wm2_ref[...], preferred_element_type=jnp.float32) \
        + bm2_ref[...]


def _tc_layer0(x, W, a8, bn=1000):
    n, d_in = x.shape
    d_h = W.shape[1]
    return pl.pallas_call(
        _tcm0_body,
        grid=(n // bn,),
        in_specs=[
            pl.BlockSpec((bn, d_in), lambda i: (i, 0)),
            pl.BlockSpec((d_in, d_h), lambda i: (0, 0)),
            pl.BlockSpec((d_h, 8), lambda i: (0, 0)),
        ],
        out_specs=[
            pl.BlockSpec((bn, d_h), lambda i: (i, 0)),
            pl.BlockSpec((bn, 8), lambda i: (i, 0)),
            pl.BlockSpec((1, 8), lambda i: (0, 0)),
        ],
        out_shape=[
            jax.ShapeDtypeStruct((n, d_h), jnp.float32),
            jax.ShapeDtypeStruct((n, 8), jnp.float32),
            jax.ShapeDtypeStruct((1, 8), jnp.float32),
        ],
    )(x, W, a8)


def _tc_layerA(p, b, W, a8, bn=1000):
    n, d_h = p.shape
    return pl.pallas_call(
        _tcmA_body,
        grid=(n // bn,),
        in_specs=[
            pl.BlockSpec((bn, d_h), lambda i: (i, 0)),
            pl.BlockSpec((1, d_h), lambda i: (0, 0)),
            pl.BlockSpec((d_h, d_h), lambda i: (0, 0)),
            pl.BlockSpec((d_h, 8), lambda i: (0, 0)),
        ],
        out_specs=[
            pl.BlockSpec((bn, d_h), lambda i: (i, 0)),
            pl.BlockSpec((bn, 8), lambda i: (i, 0)),
            pl.BlockSpec((1, 8), lambda i: (0, 0)),
        ],
        out_shape=[
            jax.ShapeDtypeStruct((n, d_h), jnp.float32),
            jax.ShapeDtypeStruct((n, 8), jnp.float32),
            jax.ShapeDtypeStruct((1, 8), jnp.float32),
        ],
    )(p, b.reshape(1, -1), W, a8)


def _tc_mlp(p, b, Wm1, bm1, Wm2, bm2, bn=1000):
    n, d_h = p.shape
    d_mlp = Wm1.shape[1]
    n_lab = Wm2.shape[1]
    return pl.pallas_call(
        _mlp_body,
        grid=(n // bn,),
        in_specs=[
            pl.BlockSpec((bn, d_h), lambda i: (i, 0)),
            pl.BlockSpec((1, d_h), lambda i: (0, 0)),
            pl.BlockSpec((d_h, d_mlp), lambda i: (0, 0)),
            pl.BlockSpec((1, d_mlp), lambda i: (0, 0)),
            pl.BlockSpec((d_mlp, n_lab), lambda i: (0, 0)),
            pl.BlockSpec((1, n_lab), lambda i: (0, 0)),
        ],
        out_specs=pl.BlockSpec((bn, n_lab), lambda i: (i, 0)),
        out_shape=jax.ShapeDtypeStruct((n, n_lab), jnp.float32),
    )(p, b.reshape(1, -1), Wm1, bm1.reshape(1, -1), Wm2, bm2.reshape(1, -1))


def _sc_mesh():
    return plsc.VectorSubcoreMesh(core_axis_name="c", subcore_axis_name="s",
                                  num_cores=NC, num_subcores=NS)


def _sc_params():
    return pltpu.CompilerParams(needs_layout_passes=False)


def _make_p1(n_edges, kchunks, mdiv_m, mdiv_s):
    ept = kchunks * 128

    @functools.partial(
        pl.kernel,
        out_type=jax.ShapeDtypeStruct((NW, 64), jnp.int32),
        mesh=_sc_mesh(),
        compiler_params=_sc_params(),
        scratch_types=[
            pltpu.VMEM((kchunks, 128), jnp.int32),
            pltpu.VMEM((64,), jnp.int32),
            pltpu.SMEM((64,), jnp.int32),
        ],
    )
    def p1(dst_hbm, c_hbm, dstb, cntv, cnt):
        cid = lax.axis_index("c")
        sid = lax.axis_index("s")
        wid = sid * NC + cid
        pltpu.sync_copy(dst_hbm.at[wid], dstb)
        for i in range(64):
            cnt[i] = 0
        iot = lax.iota(jnp.int32, L)
        ebase = wid * ept

        @pl.loop(0, kchunks)
        def _(j):
            for g in range(8):
                d = dstb[j, pl.ds(g * L, L)]
                own = (d * mdiv_m) >> mdiv_s
                eid = ebase + j * 128 + g * L + iot
                own = jnp.where(eid < n_edges, own, 32)
                for t in range(L):
                    o = own[t]
                    cnt[o] = cnt[o] + 1

        for v in range(4):
            acc = jnp.zeros((L,), jnp.int32)
            for t in range(L):
                acc = jnp.where(iot == t, cnt[v * L + t], acc)
            cntv[pl.ds(v * L, L)] = acc
        pltpu.sync_copy(cntv, c_hbm.at[wid])

    return p1


def _make_p2(n_edges, kchunks, ep_sz, mdiv_m, mdiv_s):
    ept = kchunks * 128

    @functools.partial(
        pl.kernel,
        out_type=[
            jax.ShapeDtypeStruct((ep_sz,), jnp.int32),
            jax.ShapeDtypeStruct((128,), jnp.int32),
        ],
        mesh=_sc_mesh(),
        compiler_params=_sc_params(),
        scratch_types=[
            pltpu.VMEM((kchunks, 128), jnp.int32),
            pltpu.VMEM((kchunks, 128), jnp.int32),
            pltpu.VMEM((kchunks, 128), jnp.int32),
            pltpu.VMEM((kchunks, 128), jnp.int32),
            pltpu.VMEM((NW, 64), jnp.int32),
            pltpu.VMEM((128,), jnp.int32),
            pltpu.SMEM((64,), jnp.int32),
        ],
    )
    def p2(dst_hbm, src_hbm, c_hbm, ep_hbm, meta_hbm,
           dstb, srcb, posb, pkb, cmat, metav, oloc):
        cid = lax.axis_index("c")
        sid = lax.axis_index("s")
        wid = sid * NC + cid
        pltpu.sync_copy(dst_hbm.at[wid], dstb)
        pltpu.sync_copy(src_hbm.at[wid], srcb)
        pltpu.sync_copy(c_hbm, cmat)
        iot = lax.iota(jnp.int32, L)

        tot = [jnp.zeros((L,), jnp.int32) for _ in range(4)]
        par = [jnp.zeros((L,), jnp.int32) for _ in range(4)]
        for s in range(NW):
            before = s < wid
            for v in range(4):
                row = cmat[s, pl.ds(v * L, L)]
                tot[v] = tot[v] + row
                par[v] = par[v] + jnp.where(before, row, 0)

        st = jnp.int32(0)
        starts_v = [jnp.zeros((L,), jnp.int32) for _ in range(3)]
        counts_v = [jnp.zeros((L,), jnp.int32) for _ in range(3)]
        for b in range(33):
            t_b = tot[b // L][b % L]
            p_b = par[b // L][b % L]
            oloc[b] = st + p_b
            starts_v[b // L] = jnp.where(iot == b % L, st, starts_v[b // L])
            counts_v[b // L] = jnp.where(iot == b % L, t_b, counts_v[b // L])
            st = st + (((t_b + 127) >> 7) << 7)

        ebase = wid * ept

        @pl.loop(0, kchunks)
        def _(j):
            for g in range(8):
                d = dstb[j, pl.ds(g * L, L)]
                s = srcb[j, pl.ds(g * L, L)]
                own = (d * mdiv_m) >> mdiv_s
                eid = ebase + j * 128 + g * L + iot
                own = jnp.where(eid < n_edges, own, 32)
                pkb[j, pl.ds(g * L, L)] = s * 16384 + d
                pvec = jnp.zeros((L,), jnp.int32)
                for t in range(L):
                    o = own[t]
                    p = oloc[o]
                    oloc[o] = p + 1
                    pvec = jnp.where(iot == t, p, pvec)
                posb[j, pl.ds(g * L, L)] = pvec
            pltpu.sync_copy(pkb.at[j], ep_hbm.at[posb.at[j]])

        @pl.when(wid == 0)
        def _():
            for v in range(3):
                metav[pl.ds(v * L, L)] = starts_v[v]
                metav[pl.ds(64 + v * L, L)] = counts_v[v]
            metav[pl.ds(48, L)] = jnp.zeros((L,), jnp.int32)
            metav[pl.ds(112, L)] = jnp.zeros((L,), jnp.int32)
            pltpu.sync_copy(metav, meta_hbm)

    return p2


def _make_layer(n_nodes, n_pad, rpt, d_h):
    @functools.partial(
        pl.kernel,
        out_type=jax.ShapeDtypeStruct((n_pad, d_h), jnp.float32),
        mesh=_sc_mesh(),
        compiler_params=_sc_params(),
        scratch_types=[
            pltpu.VMEM((128,), jnp.int32),
            pltpu.VMEM((128,), jnp.int32),
            pltpu.VMEM((128,), jnp.int32),
            pltpu.VMEM((128,), jnp.float32),
            pltpu.VMEM((n_nodes,), jnp.float32),
            pltpu.VMEM((n_nodes,), jnp.float32),
            pltpu.VMEM((L,), jnp.float32),
            pltpu.VMEM((128,), jnp.int32),
            pltpu.VMEM((rpt, L), jnp.float32),
            pltpu.VMEM((rpt, d_h), jnp.float32),
            pltpu.VMEM((128, d_h), jnp.float32),
        ],
    )
    def layer(ep_hbm, meta_hbm, ssrc_hbm, sdst_hbm, mvec_hbm, h_hbm,
              out_hbm,
              epbuf, srcg, dstlb, alphab, ssrc_t, sdst_t, mvec_t, metab,
              den2, acc, rowbuf):
        cid = lax.axis_index("c")
        sid = lax.axis_index("s")
        wid = sid * NC + cid
        base_node = wid * rpt

        pltpu.sync_copy(meta_hbm, metab)
        pltpu.sync_copy(ssrc_hbm, ssrc_t)
        pltpu.sync_copy(sdst_hbm, sdst_t)
        pltpu.sync_copy(mvec_hbm, mvec_t)

        iot = lax.iota(jnp.int32, L)
        widv = jnp.full((L,), wid, jnp.int32)
        mystart = plsc.load_gather(metab, [widv])[0]
        mycnt = plsc.load_gather(metab, [widv + 64])[0]
        nch = (mycnt + 127) >> 7
        mv = mvec_t[...]
        zv = jnp.zeros((L,), jnp.float32)
        ziv = jnp.zeros((L,), jnp.int32)

        @pl.loop(0, rpt)
        def _(i):
            den2[i, pl.ds(0, L)] = zv
            for q in range(d_h // L):
                acc[i, pl.ds(q * L, L)] = zv

        def edge_vec(j, g):
            pk = epbuf[pl.ds(g * L, L)]
            dg = pk & 16383
            dg = jnp.minimum(dg, n_nodes - 1)
            s = lax.shift_right_logical(pk, 14)
            s = jnp.minimum(jnp.maximum(s, 0), n_nodes - 1)
            vs = plsc.load_gather(ssrc_t, [s])
            vd = plsc.load_gather(sdst_t, [dg])
            e = _leaky(vs + vd)
            mb = _leaky(vd + mv)
            ex = jnp.exp(e - mb)
            lanei = j * 128 + g * L + iot
            ex = jnp.where(lanei < mycnt, ex, 0.0)
            dl = jnp.minimum(jnp.maximum(dg - base_node, 0), rpt - 1)
            return s, dl, ex

        lane0 = iot == 0

        @pl.loop(0, nch)
        def _(j):
            off = pl.multiple_of(mystart + j * 128, 128)
            pltpu.sync_copy(ep_hbm.at[pl.ds(off, 128)], epbuf)
            for g in range(8):
                _, dl, ex = edge_vec(j, g)
                for t in range(L):
                    oh = jnp.where(lane0, ex[t], 0.0)
                    plsc.addupdate(den2.at[dl[t], pl.ds(0, L)], oh)

        @pl.loop(0, nch)
        def _(j):
            off = pl.multiple_of(mystart + j * 128, 128)
            pltpu.sync_copy(ep_hbm.at[pl.ds(off, 128)], epbuf)
            for g in range(8):
                s, dl, ex = edge_vec(j, g)
                srcg[pl.ds(g * L, L)] = s
                dstlb[pl.ds(g * L, L)] = dl
                dv = plsc.load_gather(den2, [dl, ziv])
                alphab[pl.ds(g * L, L)] = ex / (dv + 1e-16)
            pltpu.sync_copy(h_hbm.at[srcg], rowbuf)
            for g in range(8):
                avec = alphab[pl.ds(g * L, L)]
                dvec = dstlb[pl.ds(g * L, L)]
                for t in range(L):
                    av = avec[t]
                    dl_t = dvec[t]
                    e = g * L + t
                    for q in range(d_h // L):
                        plsc.addupdate(acc.at[dl_t, pl.ds(q * L, L)],
                                       rowbuf[e, pl.ds(q * L, L)] * av)

        pltpu.sync_copy(acc, out_hbm.at[pl.ds(base_node, rpt)])

    return layer


def _a8(a_src, a_dst):
    d = a_src.shape[0]
    a8 = jnp.zeros((d, 8), jnp.float32)
    return a8.at[:, 0].set(a_src).at[:, 1].set(a_dst)


def kernel(x, edge_index, W0, a0_src, a0_dst, b0, W1, a1_src, a1_dst, b1,
           W2, a2_src, a2_dst, b2, Wm1, bm1, Wm2, bm2):
    n_nodes = x.shape[0]
    n_edges = edge_index.shape[1]
    d_h = W0.shape[1]

    rpt = 8 * (-(-n_nodes // (NW * 8)))
    n_pad = NW * rpt
    e_pad = ((n_edges + NW * 128 - 1) // (NW * 128)) * (NW * 128)
    kchunks = e_pad // (NW * 128)
    ep_sz = e_pad + NW * 128
    mdiv_m, mdiv_s = _magic_div(rpt, 16384)

    pad = e_pad - n_edges
    src3 = jnp.concatenate(
        [edge_index[0], jnp.zeros((pad,), jnp.int32)]).reshape(NW, kchunks, 128)
    dst3 = jnp.concatenate(
        [edge_index[1], jnp.zeros((pad,), jnp.int32)]).reshape(NW, kchunks, 128)

    p1_fn = _make_p1(n_edges, kchunks, mdiv_m, mdiv_s)
    p2_fn = _make_p2(n_edges, kchunks, ep_sz, mdiv_m, mdiv_s)
    layer_fn = _make_layer(n_nodes, n_pad, rpt, d_h)

    cmat = p1_fn(dst3)
    ep, meta = p2_fn(dst3, src3, cmat)

    def edge_phase(h, s8, mstat):
        s_src = s8[:, 0]
        s_dst = s8[:, 1]
        mvec = jnp.full((L,), mstat[0, 0], jnp.float32)
        out = layer_fn(ep, meta, s_src, s_dst, mvec, h)
        return out[:n_nodes]

    h, s8, mstat = _tc_layer0(x, W0, _a8(a0_src, a0_dst))
    p = edge_phase(h, s8, mstat)
    h, s8, mstat = _tc_layerA(p, b0, W1, _a8(a1_src, a1_dst))
    p = edge_phase(h, s8, mstat)
    h, s8, mstat = _tc_layerA(p, b1, W2, _a8(a2_src, a2_dst))
    p = edge_phase(h, s8, mstat)
    return _tc_mlp(p, b2, Wm1, bm1, Wm2, bm2)
